# baseline (speedup 1.0000x reference)
import functools
import jax
import jax.numpy as jnp
from jax import lax
from jax.experimental import pallas as pl
from jax.experimental.pallas import tpu as pltpu
from jax.experimental.pallas import tpu_sc as plsc

B = 16384
D = 128
L = 16
NW = 32
B_PER_W = B // NW
CH = 128
N_CHUNK = B_PER_W // CH


def _bpr_body(user_hbm, pos_hbm, neg_hbm, utab_hbm, itab_hbm, out_hbm,
              idx_u, idx_i, idx_j,
              ru0, ri0, rj0, ru1, ri1, rj1, out_v,
              su0, si0, sj0, su1, si1, sj1, so):
    wid = lax.axis_index("s") * 2 + lax.axis_index("c")
    base = wid * B_PER_W

    cxu = pltpu.async_copy(user_hbm.at[pl.ds(base, B_PER_W)], idx_u, so)
    cxi = pltpu.async_copy(pos_hbm.at[pl.ds(base, B_PER_W)], idx_i, su1)
    cxj = pltpu.async_copy(neg_hbm.at[pl.ds(base, B_PER_W)], idx_j, si1)

    bufs = [(ru0, ri0, rj0, su0, si0, sj0), (ru1, ri1, rj1, su1, si1, sj1)]
    chunks = [(0, 128), (128, 128), (256, 128), (384, 128)]

    def start(k):
        off, n = chunks[k]
        ru, ri, rj, su, si, sj = bufs[k % 2]
        cu = pltpu.async_copy(utab_hbm.at[idx_u.at[pl.ds(off, n)]],
                              ru.at[pl.ds(0, n)], su)
        ci = pltpu.async_copy(itab_hbm.at[idx_i.at[pl.ds(off, n)]],
                              ri.at[pl.ds(0, n)], si)
        cj = pltpu.async_copy(itab_hbm.at[idx_j.at[pl.ds(off, n)]],
                              rj.at[pl.ds(0, n)], sj)
        return (cu, ci, cj)

    lane = lax.iota(jnp.int32, L)
    ru, ri, rj, su, si, sj = bufs[0]
    n0 = chunks[0][1]
    cxu.wait()
    c0u = pltpu.async_copy(utab_hbm.at[idx_u.at[pl.ds(0, n0)]],
                           ru.at[pl.ds(0, n0)], su)
    cxi.wait()
    c0i = pltpu.async_copy(itab_hbm.at[idx_i.at[pl.ds(0, n0)]],
                           ri.at[pl.ds(0, n0)], si)
    cxj.wait()
    c0j = pltpu.async_copy(itab_hbm.at[idx_j.at[pl.ds(0, n0)]],
                           rj.at[pl.ds(0, n0)], sj)
    pending = (c0u, c0i, c0j)
    out_pending = []
    for k in range(len(chunks)):
        off, n = chunks[k]
        ru, ri, rj = bufs[k % 2][:3]
        nxt = start(k + 1) if k + 1 < len(chunks) else None
        for c in pending:
            c.wait()
        pending = nxt

        @plsc.parallel_loop(0, n // L, unroll=2)
        def group_body(g, ru=ru, ri=ri, rj=rj, off=off):
            def row_body(r16, res):
                r = g * L + r16
                acc = jnp.zeros((L,), jnp.float32)
                for c in range(D // L):
                    uv = ru[r, pl.ds(c * L, L)]
                    iv = ri[r, pl.ds(c * L, L)]
                    jv = rj[r, pl.ds(c * L, L)]
                    acc = acc + uv * (iv - jv)
                return jnp.where(lane == r16, jnp.sum(acc), res)

            res = lax.fori_loop(0, L, row_body, jnp.zeros((L,), jnp.float32),
                                unroll=4)
            out_v[pl.ds(off + g * L, L)] = res

    del out_pending
    pltpu.async_copy(out_v, out_hbm.at[pl.ds(base, B_PER_W)], so).wait()


@jax.jit
def _bpr(user, pos_item, neg_item, user_table, item_table):
    mesh = plsc.VectorSubcoreMesh(core_axis_name="c", subcore_axis_name="s")
    f = functools.partial(
        pl.kernel,
        mesh=mesh,
        compiler_params=pltpu.CompilerParams(
            needs_layout_passes=False, disable_bounds_checks=True),
        out_type=jax.ShapeDtypeStruct((B,), jnp.float32),
        scratch_types=[
            pltpu.VMEM((B_PER_W,), jnp.int32),
            pltpu.VMEM((B_PER_W,), jnp.int32),
            pltpu.VMEM((B_PER_W,), jnp.int32),
            pltpu.VMEM((CH, D), jnp.float32),
            pltpu.VMEM((CH, D), jnp.float32),
            pltpu.VMEM((CH, D), jnp.float32),
            pltpu.VMEM((CH, D), jnp.float32),
            pltpu.VMEM((CH, D), jnp.float32),
            pltpu.VMEM((CH, D), jnp.float32),
            pltpu.VMEM((B_PER_W,), jnp.float32),
            pltpu.SemaphoreType.DMA,
            pltpu.SemaphoreType.DMA,
            pltpu.SemaphoreType.DMA,
            pltpu.SemaphoreType.DMA,
            pltpu.SemaphoreType.DMA,
            pltpu.SemaphoreType.DMA,
            pltpu.SemaphoreType.DMA,
        ],
    )(_bpr_body)
    return f(user, pos_item, neg_item, user_table, item_table)


def kernel(user, pos_item, neg_item, user_table, item_table):
    return _bpr(user, pos_item, neg_item, user_table, item_table)


# speedup vs baseline: 1.1752x; 1.1752x over previous
import functools
import jax
import jax.numpy as jnp
from jax import lax
from jax.experimental import pallas as pl
from jax.experimental.pallas import tpu as pltpu
from jax.experimental.pallas import tpu_sc as plsc

B = 16384
D = 128
L = 16
NW = 32
B_PER_W = B // NW
CH = 128
N_CHUNK = B_PER_W // CH


def _bpr_body(user_hbm, pos_hbm, neg_hbm, utab_hbm, itab_hbm, out_hbm,
              idx_u, idx_i, idx_j,
              ru0, ri0, rj0, ru1, ri1, rj1, out_v,
              su0, si0, sj0, su1, si1, sj1, so):
    wid = lax.axis_index("s") * 2 + lax.axis_index("c")
    base = wid * B_PER_W

    cxu = pltpu.async_copy(user_hbm.at[pl.ds(base, B_PER_W)], idx_u, so)
    cxi = pltpu.async_copy(pos_hbm.at[pl.ds(base, B_PER_W)], idx_i, su1)
    cxj = pltpu.async_copy(neg_hbm.at[pl.ds(base, B_PER_W)], idx_j, si1)

    bufs = [(ru0, ri0, rj0, su0, si0, sj0), (ru1, ri1, rj1, su1, si1, sj1)]
    chunks = [(0, 128), (128, 128), (256, 128), (384, 128)]

    def start(k):
        off, n = chunks[k]
        ru, ri, rj, su, si, sj = bufs[k % 2]
        cu = pltpu.async_copy(utab_hbm.at[idx_u.at[pl.ds(off, n)]],
                              ru.at[pl.ds(0, n)], su)
        ci = pltpu.async_copy(itab_hbm.at[idx_i.at[pl.ds(off, n)]],
                              ri.at[pl.ds(0, n)], si)
        cj = pltpu.async_copy(itab_hbm.at[idx_j.at[pl.ds(off, n)]],
                              rj.at[pl.ds(0, n)], sj)
        return (cu, ci, cj)

    lane = lax.iota(jnp.int32, L)
    ru, ri, rj, su, si, sj = bufs[0]
    n0 = chunks[0][1]
    cxu.wait()
    c0u = pltpu.async_copy(utab_hbm.at[idx_u.at[pl.ds(0, n0)]],
                           ru.at[pl.ds(0, n0)], su)
    cxi.wait()
    c0i = pltpu.async_copy(itab_hbm.at[idx_i.at[pl.ds(0, n0)]],
                           ri.at[pl.ds(0, n0)], si)
    cxj.wait()
    c0j = pltpu.async_copy(itab_hbm.at[idx_j.at[pl.ds(0, n0)]],
                           rj.at[pl.ds(0, n0)], sj)
    start(1)

    def wait_chunk(b):
        ru, ri, rj, su, si, sj = bufs[b]
        pltpu.make_async_copy(utab_hbm.at[idx_u.at[pl.ds(0, CH)]], ru, su).wait()
        pltpu.make_async_copy(itab_hbm.at[idx_i.at[pl.ds(0, CH)]], ri, si).wait()
        pltpu.make_async_copy(itab_hbm.at[idx_j.at[pl.ds(0, CH)]], rj, sj).wait()

    def issue(k_dyn, b):
        ru, ri, rj, su, si, sj = bufs[b]
        off = k_dyn * CH
        pltpu.async_copy(utab_hbm.at[idx_u.at[pl.ds(off, CH)]], ru, su)
        pltpu.async_copy(itab_hbm.at[idx_i.at[pl.ds(off, CH)]], ri, si)
        pltpu.async_copy(itab_hbm.at[idx_j.at[pl.ds(off, CH)]], rj, sj)

    def compute(b, off):
        ru, ri, rj = bufs[b][:3]

        @plsc.parallel_loop(0, CH // L)
        def group_body(g):
            def row_body(r16, res):
                r = g * L + r16
                acc = jnp.zeros((L,), jnp.float32)
                for c in range(D // L):
                    uv = ru[r, pl.ds(c * L, L)]
                    iv = ri[r, pl.ds(c * L, L)]
                    jv = rj[r, pl.ds(c * L, L)]
                    acc = acc + uv * (iv - jv)
                return jnp.where(lane == r16, jnp.sum(acc), res)

            res = lax.fori_loop(0, L, row_body, jnp.zeros((L,), jnp.float32),
                                unroll=4)
            out_v[pl.ds(off + g * L, L)] = res

    def pair_body(t, _):
        wait_chunk(0)

        @pl.when(t < N_CHUNK // 2 - 1)
        def _():
            issue(2 * t + 2, 0)

        compute(0, 2 * t * CH)
        wait_chunk(1)

        @pl.when(t < N_CHUNK // 2 - 1)
        def _():
            issue(2 * t + 3, 1)

        compute(1, (2 * t + 1) * CH)
        return 0

    lax.fori_loop(0, N_CHUNK // 2, pair_body, 0)
    pltpu.async_copy(out_v, out_hbm.at[pl.ds(base, B_PER_W)], so).wait()


@jax.jit
def _bpr(user, pos_item, neg_item, user_table, item_table):
    mesh = plsc.VectorSubcoreMesh(core_axis_name="c", subcore_axis_name="s")
    f = functools.partial(
        pl.kernel,
        mesh=mesh,
        compiler_params=pltpu.CompilerParams(
            needs_layout_passes=False, disable_bounds_checks=True),
        out_type=jax.ShapeDtypeStruct((B,), jnp.float32),
        scratch_types=[
            pltpu.VMEM((B_PER_W,), jnp.int32),
            pltpu.VMEM((B_PER_W,), jnp.int32),
            pltpu.VMEM((B_PER_W,), jnp.int32),
            pltpu.VMEM((CH, D), jnp.float32),
            pltpu.VMEM((CH, D), jnp.float32),
            pltpu.VMEM((CH, D), jnp.float32),
            pltpu.VMEM((CH, D), jnp.float32),
            pltpu.VMEM((CH, D), jnp.float32),
            pltpu.VMEM((CH, D), jnp.float32),
            pltpu.VMEM((B_PER_W,), jnp.float32),
            pltpu.SemaphoreType.DMA,
            pltpu.SemaphoreType.DMA,
            pltpu.SemaphoreType.DMA,
            pltpu.SemaphoreType.DMA,
            pltpu.SemaphoreType.DMA,
            pltpu.SemaphoreType.DMA,
            pltpu.SemaphoreType.DMA,
        ],
    )(_bpr_body)
    return f(user, pos_item, neg_item, user_table, item_table)


def kernel(user, pos_item, neg_item, user_table, item_table):
    return _bpr(user, pos_item, neg_item, user_table, item_table)
